# baseline (speedup 1.0000x reference)
import functools

import jax
import jax.numpy as jnp
from jax import lax
from jax.experimental import pallas as pl
from jax.experimental.pallas import tpu as pltpu
from jax.experimental.pallas import tpu_sc as plsc

_NC, _NS = 2, 16
_NW = _NC * _NS
_LANES = 16
_NBUF = 4


def _pos_body(row_ref, col_ref, pos_ref):
    row_t = row_ref[...].T
    col_t = col_ref[...].T
    pos3 = row_t[:, :, None] + col_t[:, None, :]
    pos_ref[...] = pos3.reshape(pos_ref.shape)


def _build_pos(row_embed, col_embed):
    H, D = row_embed.shape
    W = col_embed.shape[0]
    return pl.pallas_call(
        _pos_body,
        out_shape=jax.ShapeDtypeStruct((D, H * W), jnp.float32),
    )(row_embed, col_embed)


def _sc_add_kernel(B, D, HW):
    d_per_w = D // _NW
    rows = d_per_w // 2
    n_chunks = 2 * B

    mesh = plsc.VectorSubcoreMesh(core_axis_name="c", subcore_axis_name="s")

    @functools.partial(
        pl.kernel,
        out_type=jax.ShapeDtypeStruct((B, D, HW), jnp.float32),
        mesh=mesh,
        scratch_types=[
            pltpu.VMEM((d_per_w, HW), jnp.float32),
            pltpu.VMEM((_NBUF, rows, HW), jnp.float32),
            pltpu.SemaphoreType.DMA((_NBUF,)),
            pltpu.SemaphoreType.DMA((_NBUF,)),
        ],
    )
    def k(x_hbm, pos_hbm, out_hbm, pos_v, bufs, sis, sos):
        w = lax.axis_index("s") * _NC + lax.axis_index("c")
        d0 = w * d_per_w

        def in_copy(t):
            b, half, s = t // 2, t % 2, t % _NBUF
            return pltpu.make_async_copy(
                x_hbm.at[b, pl.ds(d0 + half * rows, rows)],
                bufs.at[s], sis.at[s])

        def out_copy(t):
            b, half, s = t // 2, t % 2, t % _NBUF
            return pltpu.make_async_copy(
                bufs.at[s],
                out_hbm.at[b, pl.ds(d0 + half * rows, rows)], sos.at[s])

        in_copy(0).start()
        in_copy(1).start()
        pltpu.sync_copy(pos_hbm.at[pl.ds(d0, d_per_w)], pos_v)

        def step(t, carry):
            s = t % _NBUF
            half = t % 2
            in_copy(t).wait()

            for r in range(rows):
                def _body(i, r=r, s=s, half=half):
                    sl = pl.ds(i, _LANES)
                    plsc.addupdate(bufs.at[s, r, sl],
                                   pos_v[half * rows + r, sl])

                plsc.parallel_loop(0, HW, step=16, unroll=8)(_body)

            out_copy(t).start()

            @pl.when(t + 2 < n_chunks)
            def _():
                @pl.when(t >= 2)
                def _():
                    out_copy(t - 2).wait()
                in_copy(t + 2).start()

            return carry

        lax.fori_loop(0, n_chunks, step, 0)
        for t in range(n_chunks - _NBUF, n_chunks):
            out_copy(t).wait()

    return k


def kernel(x, row_embed, col_embed):
    B, D, H, W = x.shape
    HW = H * W
    G = 4
    Bg = B // G
    pos = _build_pos(row_embed, col_embed)
    xf = x.reshape(B, D, HW)
    sc = _sc_add_kernel(Bg, D, HW)
    outs = [sc(xf[g * Bg:(g + 1) * Bg], pos) for g in range(G)]
    out = jnp.concatenate(outs, axis=0)
    return out.reshape(B, D, H, W)


# speedup vs baseline: 1.3967x; 1.3967x over previous
import functools

import jax
import jax.numpy as jnp
from jax import lax
from jax.experimental import pallas as pl
from jax.experimental.pallas import tpu as pltpu
from jax.experimental.pallas import tpu_sc as plsc

_NC, _NS = 2, 16
_NW = _NC * _NS
_LANES = 16
_NBUF = 4


def _pos_body(row_ref, col_ref, pos_ref):
    row_t = row_ref[...].T
    col_t = col_ref[...].T
    pos3 = row_t[:, :, None] + col_t[:, None, :]
    pos_ref[...] = pos3.reshape(pos_ref.shape)


def _build_pos(row_embed, col_embed):
    H, D = row_embed.shape
    W = col_embed.shape[0]
    return pl.pallas_call(
        _pos_body,
        out_shape=jax.ShapeDtypeStruct((D, H * W), jnp.float32),
    )(row_embed, col_embed)


def _sc_add_kernel(B, D, HW):
    d_per_w = D // _NW
    rows = d_per_w // 2
    n_chunks = 2 * B

    mesh = plsc.VectorSubcoreMesh(core_axis_name="c", subcore_axis_name="s")

    @functools.partial(
        pl.kernel,
        out_type=jax.ShapeDtypeStruct((B, D, HW), jnp.float32),
        mesh=mesh,
        scratch_types=[
            pltpu.VMEM((d_per_w, HW), jnp.float32),
            pltpu.VMEM((_NBUF, rows, HW), jnp.float32),
            pltpu.SemaphoreType.DMA((_NBUF,)),
            pltpu.SemaphoreType.DMA((_NBUF,)),
        ],
    )
    def k(x_hbm, pos_hbm, out_hbm, pos_v, bufs, sis, sos):
        w = lax.axis_index("s") * _NC + lax.axis_index("c")
        d0 = w * d_per_w

        def in_copy(t):
            b, half, s = t // 2, t % 2, t % _NBUF
            return pltpu.make_async_copy(
                x_hbm.at[b, pl.ds(d0 + half * rows, rows)],
                bufs.at[s], sis.at[s])

        def out_copy(t):
            b, half, s = t // 2, t % 2, t % _NBUF
            return pltpu.make_async_copy(
                bufs.at[s],
                out_hbm.at[b, pl.ds(d0 + half * rows, rows)], sos.at[s])

        in_copy(0).start()
        in_copy(1).start()
        pltpu.sync_copy(pos_hbm.at[pl.ds(d0, d_per_w)], pos_v)

        def step(t, carry):
            s = t % _NBUF
            half = t % 2
            in_copy(t).wait()

            for r in range(rows):
                def _body(i, r=r, s=s, half=half):
                    sl = pl.ds(i, _LANES)
                    plsc.addupdate(bufs.at[s, r, sl],
                                   pos_v[half * rows + r, sl])

                plsc.parallel_loop(0, HW, step=16, unroll=8)(_body)

            out_copy(t).start()

            @pl.when(t + 2 < n_chunks)
            def _():
                @pl.when(t >= 2)
                def _():
                    out_copy(t - 2).wait()
                in_copy(t + 2).start()

            return carry

        lax.fori_loop(0, n_chunks, step, 0)
        for t in range(n_chunks - _NBUF, n_chunks):
            out_copy(t).wait()

    return k


def kernel(x, row_embed, col_embed):
    B, D, H, W = x.shape
    HW = H * W
    pos = _build_pos(row_embed, col_embed)
    xf = x.reshape(B, D, HW)
    out = _sc_add_kernel(B, D, HW)(xf, pos)
    return out.reshape(B, D, H, W)
